# baseline (speedup 1.0000x reference)
import jax
import jax.numpy as jnp
from jax import lax
from jax.experimental import pallas as pl
from jax.experimental.pallas import tpu as pltpu
from jax.experimental.pallas import tpu_sc as plsc

NC = 2
NS = 16
L = 16
NW = NC * NS
BATCH = 16384
BPW = BATCH // NW
D = 16


def _mf_body(uid_hbm, iid_hbm, p_hbm, q_hbm, out_hbm,
             idx_u, idx_i, pu, qi, out_v, sem_u, sem_i):
    wid = lax.axis_index("s") * NC + lax.axis_index("c")
    base = wid * BPW

    pltpu.sync_copy(uid_hbm.at[pl.ds(base, BPW)], idx_u)
    pltpu.sync_copy(iid_hbm.at[pl.ds(base, BPW)], idx_i)

    cp_p = pltpu.async_copy(p_hbm.at[idx_u], pu, sem_u)
    cp_q = pltpu.async_copy(q_hbm.at[idx_i], qi, sem_i)
    cp_p.wait()
    cp_q.wait()

    rows0 = lax.iota(jnp.int32, L)

    def blk_body(b, carry):
        rows = rows0 + b * L
        acc = jnp.zeros((L,), jnp.float32)
        for d in range(D):
            cols = jnp.full((L,), d, jnp.int32)
            vp = plsc.load_gather(pu, [rows, cols])
            vq = plsc.load_gather(qi, [rows, cols])
            acc = acc + vp * vq
        out_v[pl.ds(pl.multiple_of(b * L, L), L)] = acc
        return carry

    lax.fori_loop(0, BPW // L, blk_body, 0)

    pltpu.sync_copy(out_v, out_hbm.at[pl.ds(base, BPW)])


def kernel(user_id, item_id, P, Q):
    uid = user_id.astype(jnp.int32)
    iid = item_id.astype(jnp.int32)
    mesh = plsc.VectorSubcoreMesh(core_axis_name="c", subcore_axis_name="s")
    out = pl.kernel(
        _mf_body,
        out_type=jax.ShapeDtypeStruct((BATCH,), jnp.float32),
        mesh=mesh,
        compiler_params=pltpu.CompilerParams(
            needs_layout_passes=False, use_tc_tiling_on_sc=False),
        scratch_types=[
            pltpu.VMEM((BPW,), jnp.int32),
            pltpu.VMEM((BPW,), jnp.int32),
            pltpu.VMEM((BPW, D), jnp.float32),
            pltpu.VMEM((BPW, D), jnp.float32),
            pltpu.VMEM((BPW,), jnp.float32),
            pltpu.SemaphoreType.DMA,
            pltpu.SemaphoreType.DMA,
        ],
    )(uid, iid, P, Q)
    return out.reshape(-1, 1)


# speedup vs baseline: 5.3002x; 5.3002x over previous
import jax
import jax.numpy as jnp
from jax import lax
from jax.experimental import pallas as pl
from jax.experimental.pallas import tpu as pltpu
from jax.experimental.pallas import tpu_sc as plsc

NC = 2
NS = 16
L = 16
NW = NC * NS
BATCH = 16384
BPW = BATCH // NW
D = 16
NBUF = 16

def _slot_col_idx(k, c):
    rows = lax.iota(jnp.int32, L)
    cols = jnp.full((L,), k * 128, jnp.int32) + c
    return rows, cols


def _mf_body(uid_hbm, iid_hbm, pt_hbm, qt_hbm, out_hbm,
             idx_u, idx_i, ring_p, ring_q, pu, qi, out_v,
             sem_p, sem_q):
    wid = lax.axis_index("s") * NC + lax.axis_index("c")
    base = wid * BPW

    pltpu.sync_copy(uid_hbm.at[pl.ds(base, BPW)], idx_u)
    pltpu.sync_copy(iid_hbm.at[pl.ds(base, BPW)], idx_i)

    def grp_body(g, carry):
        gbase = pl.multiple_of(g * L, L)
        u_vec = idx_u[pl.ds(gbase, L)]
        i_vec = idx_i[pl.ds(gbase, L)]
        for l in range(L):
            ub = pl.multiple_of((u_vec[l] // 128) * 128, 128)
            ib = pl.multiple_of((i_vec[l] // 128) * 128, 128)
            ko = pl.multiple_of(l * 128, 128)
            pltpu.async_copy(pt_hbm.at[:, pl.ds(ub, 128)],
                             ring_p.at[:, pl.ds(ko, 128)], sem_p)
            pltpu.async_copy(qt_hbm.at[:, pl.ds(ib, 128)],
                             ring_q.at[:, pl.ds(ko, 128)], sem_q)
        for l in range(L):
            pltpu.make_async_copy(pt_hbm.at[:, pl.ds(0, 128)],
                                  ring_p.at[:, pl.ds(0, 128)], sem_p).wait()
            pltpu.make_async_copy(qt_hbm.at[:, pl.ds(0, 128)],
                                  ring_q.at[:, pl.ds(0, 128)], sem_q).wait()
        for l in range(L):
            off = pl.multiple_of((g * L + l) * D, D)
            ru, cu = _slot_col_idx(l, u_vec[l] % 128)
            ri, ci = _slot_col_idx(l, i_vec[l] % 128)
            pu[pl.ds(off, D)] = plsc.load_gather(ring_p, [ru, cu])
            qi[pl.ds(off, D)] = plsc.load_gather(ring_q, [ri, ci])
        return carry

    lax.fori_loop(0, BPW // L, grp_body, 0)

    flat0 = lax.iota(jnp.int32, L) * D

    def blk_body(b, carry):
        flat = flat0 + b * (L * D)
        acc = jnp.zeros((L,), jnp.float32)
        for d in range(D):
            vp = plsc.load_gather(pu, [flat + d])
            vq = plsc.load_gather(qi, [flat + d])
            acc = acc + vp * vq
        out_v[pl.ds(pl.multiple_of(b * L, L), L)] = acc
        return carry

    lax.fori_loop(0, BPW // L, blk_body, 0)

    pltpu.sync_copy(out_v, out_hbm.at[pl.ds(base, BPW)])


def kernel(user_id, item_id, P, Q):
    uid = user_id.astype(jnp.int32)
    iid = item_id.astype(jnp.int32)
    mesh = plsc.VectorSubcoreMesh(core_axis_name="c", subcore_axis_name="s")
    out = pl.kernel(
        _mf_body,
        out_type=jax.ShapeDtypeStruct((BATCH,), jnp.float32),
        mesh=mesh,
        compiler_params=pltpu.CompilerParams(
            needs_layout_passes=False, use_tc_tiling_on_sc=True),
        scratch_types=[
            pltpu.VMEM((BPW,), jnp.int32),
            pltpu.VMEM((BPW,), jnp.int32),
            pltpu.VMEM((D, NBUF * 128), jnp.float32),
            pltpu.VMEM((D, NBUF * 128), jnp.float32),
            pltpu.VMEM((BPW * D,), jnp.float32),
            pltpu.VMEM((BPW * D,), jnp.float32),
            pltpu.VMEM((BPW,), jnp.float32),
            pltpu.SemaphoreType.DMA,
            pltpu.SemaphoreType.DMA,
        ],
    )(uid, iid, P.T, Q.T)
    return out.reshape(-1, 1)


# speedup vs baseline: 6.1080x; 1.1524x over previous
import jax
import jax.numpy as jnp
from jax import lax
from jax.experimental import pallas as pl
from jax.experimental.pallas import tpu as pltpu
from jax.experimental.pallas import tpu_sc as plsc

NC = 2
NS = 16
L = 16
NW = NC * NS
BATCH = 16384
BPW = BATCH // NW
D = 16
NBUF = 16

def _slot_col_idx(k, c):
    rows = lax.iota(jnp.int32, L)
    cols = jnp.full((L,), k * 128, jnp.int32) + c
    return rows, cols


def _mf_body(uid_hbm, iid_hbm, pt_hbm, qt_hbm, out_hbm,
             idx_u, idx_i, ring_p, ring_q, pu, qi, out_v,
             sem_pa, sem_pb, sem_qa, sem_qb):
    wid = lax.axis_index("s") * NC + lax.axis_index("c")
    base = wid * BPW

    pltpu.sync_copy(uid_hbm.at[pl.ds(base, BPW)], idx_u)
    pltpu.sync_copy(iid_hbm.at[pl.ds(base, BPW)], idx_i)

    H = L // 2

    def fire_half(vecs, half):
        u_vec, i_vec = vecs
        sp = sem_pa if half == 0 else sem_pb
        sq = sem_qa if half == 0 else sem_qb
        for l in range(H):
            ub = pl.multiple_of((u_vec[half * H + l] // 128) * 128, 128)
            ib = pl.multiple_of((i_vec[half * H + l] // 128) * 128, 128)
            ko = pl.multiple_of((half * H + l) * 128, 128)
            pltpu.async_copy(pt_hbm.at[:, pl.ds(ub, 128)],
                             ring_p.at[:, pl.ds(ko, 128)], sp)
            pltpu.async_copy(qt_hbm.at[:, pl.ds(ib, 128)],
                             ring_q.at[:, pl.ds(ko, 128)], sq)

    def drain_half(half):
        sp = sem_pa if half == 0 else sem_pb
        sq = sem_qa if half == 0 else sem_qb
        for _ in range(H):
            pltpu.make_async_copy(pt_hbm.at[:, pl.ds(0, 128)],
                                  ring_p.at[:, pl.ds(0, 128)], sp).wait()
            pltpu.make_async_copy(qt_hbm.at[:, pl.ds(0, 128)],
                                  ring_q.at[:, pl.ds(0, 128)], sq).wait()

    def extract_half(g, vecs, half):
        u_vec, i_vec = vecs
        for l in range(H):
            off = pl.multiple_of((g * L + half * H + l) * D, D)
            ru, cu = _slot_col_idx(half * H + l, u_vec[half * H + l] % 128)
            ri, ci = _slot_col_idx(half * H + l, i_vec[half * H + l] % 128)
            pu[pl.ds(off, D)] = plsc.load_gather(ring_p, [ru, cu])
            qi[pl.ds(off, D)] = plsc.load_gather(ring_q, [ri, ci])

    def load_vecs(g):
        gc = jnp.minimum(g, BPW // L - 1)
        gbase = pl.multiple_of(gc * L, L)
        return idx_u[pl.ds(gbase, L)], idx_i[pl.ds(gbase, L)]

    vecs0 = load_vecs(0)
    fire_half(vecs0, 0)

    def grp_body(g, carry):
        vecs = load_vecs(g)
        fire_half(vecs, 1)
        drain_half(0)
        extract_half(g, vecs, 0)
        nvecs = load_vecs(g + 1)

        @pl.when(g + 1 < BPW // L)
        def _():
            fire_half(nvecs, 0)
        drain_half(1)
        extract_half(g, vecs, 1)
        return carry

    lax.fori_loop(0, BPW // L, grp_body, 0)

    flat0 = lax.iota(jnp.int32, L) * D

    def blk_body(b, carry):
        flat = flat0 + b * (L * D)
        acc = jnp.zeros((L,), jnp.float32)
        for d in range(D):
            vp = plsc.load_gather(pu, [flat + d])
            vq = plsc.load_gather(qi, [flat + d])
            acc = acc + vp * vq
        out_v[pl.ds(pl.multiple_of(b * L, L), L)] = acc
        return carry

    lax.fori_loop(0, BPW // L, blk_body, 0)

    pltpu.sync_copy(out_v, out_hbm.at[pl.ds(base, BPW)])


def kernel(user_id, item_id, P, Q):
    uid = user_id.astype(jnp.int32)
    iid = item_id.astype(jnp.int32)
    mesh = plsc.VectorSubcoreMesh(core_axis_name="c", subcore_axis_name="s")
    out = pl.kernel(
        _mf_body,
        out_type=jax.ShapeDtypeStruct((BATCH,), jnp.float32),
        mesh=mesh,
        compiler_params=pltpu.CompilerParams(
            needs_layout_passes=False, use_tc_tiling_on_sc=True),
        scratch_types=[
            pltpu.VMEM((BPW,), jnp.int32),
            pltpu.VMEM((BPW,), jnp.int32),
            pltpu.VMEM((D, NBUF * 128), jnp.float32),
            pltpu.VMEM((D, NBUF * 128), jnp.float32),
            pltpu.VMEM((BPW * D,), jnp.float32),
            pltpu.VMEM((BPW * D,), jnp.float32),
            pltpu.VMEM((BPW,), jnp.float32),
            pltpu.SemaphoreType.DMA,
            pltpu.SemaphoreType.DMA,
            pltpu.SemaphoreType.DMA,
            pltpu.SemaphoreType.DMA,
        ],
    )(uid, iid, P.T, Q.T)
    return out.reshape(-1, 1)
